# Initial kernel scaffold; baseline (speedup 1.0000x reference)
import functools
import math

import jax
import jax.numpy as jnp
import numpy as np
from jax.experimental import pallas as pl

H_IMG = 256
W_IMG = 256
HF = 32
WF = 32
STRIDE = 8
A = 3
PRE = 1000
POST = 100
NMS_T = 0.7
N_SC = HF * WF * A
NP = 1024
DWH_CLIP = math.log(1000.0 / 16)


def _make_anchors_np():
    ratios = np.array([0.5, 1.0, 2.0])
    hr = np.sqrt(ratios)
    wr = 1.0 / hr
    size = float(min(HF, WF))
    ws = wr * size
    hs = hr * size
    base = np.stack([-ws / 2, -hs / 2, ws / 2, hs / 2], axis=1)
    sy, sx = np.meshgrid(np.arange(HF) * STRIDE, np.arange(WF) * STRIDE,
                         indexing='ij')
    shifts = np.stack([sx.ravel(), sy.ravel(), sx.ravel(), sy.ravel()], axis=1)
    anchors = (shifts[:, None, :] + base[None, :, :]).reshape(-1, 4)
    return anchors.astype(np.float32)


def _mm_body(a_ref, b_ref, bias_ref, o_ref, *, relu):
    acc = jnp.dot(a_ref[...], b_ref[...], preferred_element_type=jnp.float32)
    acc = acc + bias_ref[...]
    if relu:
        acc = jnp.maximum(acc, 0.0)
    o_ref[...] = acc


def _matmul(a, b, bias, relu):
    m, _ = a.shape
    n = b.shape[1]
    return pl.pallas_call(
        functools.partial(_mm_body, relu=relu),
        out_shape=jax.ShapeDtypeStruct((m, n), jnp.float32),
    )(a, b, bias.reshape(1, n))


def _im2col(x, stride, pad):
    c, h, w = x.shape
    xp = jnp.pad(x, ((0, 0), pad[0], pad[1]))
    oh = (h + pad[0][0] + pad[0][1] - 3) // stride + 1
    ow = (w + pad[1][0] + pad[1][1] - 3) // stride + 1
    cols = []
    for ky in range(3):
        for kx in range(3):
            sl = jax.lax.slice(
                xp, (0, ky, kx),
                (c, ky + (oh - 1) * stride + 1, kx + (ow - 1) * stride + 1),
                (1, stride, stride))
            cols.append(sl)
    p = jnp.stack(cols, axis=0)
    p = p.transpose(2, 3, 1, 0)
    return p.reshape(oh * ow, c * 9)


def _decode_boxes(ax1, ay1, ax2, ay2, dx, dy, dw, dh):
    aw = ax2 - ax1
    ah = ay2 - ay1
    acx = ax1 + 0.5 * aw
    acy = ay1 + 0.5 * ah
    dw = jnp.minimum(dw, DWH_CLIP)
    dh = jnp.minimum(dh, DWH_CLIP)
    pcx = dx * aw + acx
    pcy = dy * ah + acy
    pw = jnp.exp(dw) * aw
    ph = jnp.exp(dh) * ah
    x1 = jnp.clip(pcx - 0.5 * pw, 0.0, float(W_IMG))
    y1 = jnp.clip(pcy - 0.5 * ph, 0.0, float(H_IMG))
    x2 = jnp.clip(pcx + 0.5 * pw, 0.0, float(W_IMG))
    y2 = jnp.clip(pcy + 0.5 * ph, 0.0, float(H_IMG))
    return x1, y1, x2, y2


def _proposal_body(s_row_ref, s_col_ref, d8_ref, d8t_ref, o_ref):
    s_row = s_row_ref[...]
    s_col = s_col_ref[...]
    f32 = jnp.float32

    CH = 512
    rank_row = jnp.zeros((1, N_SC), f32)
    rank_col = jnp.zeros((N_SC, 1), f32)
    for c in range(N_SC // CH):
        sk_col = jax.lax.slice(s_col, (c * CH, 0), ((c + 1) * CH, 1))
        kk = jax.lax.broadcasted_iota(jnp.int32, (CH, N_SC), 0) + c * CH
        jj = jax.lax.broadcasted_iota(jnp.int32, (CH, N_SC), 1)
        beat = (sk_col > s_row) | ((sk_col == s_row) & (kk < jj))
        rank_row = rank_row + jnp.sum(beat.astype(f32), axis=0, keepdims=True)

        sk_row = jax.lax.slice(s_row, (0, c * CH), (1, (c + 1) * CH))
        kk2 = jax.lax.broadcasted_iota(jnp.int32, (N_SC, CH), 1) + c * CH
        ii2 = jax.lax.broadcasted_iota(jnp.int32, (N_SC, CH), 0)
        beat2 = (sk_row > s_col) | ((sk_row == s_col) & (kk2 < ii2))
        rank_col = rank_col + jnp.sum(beat2.astype(f32), axis=1, keepdims=True)

    r_col = jax.lax.broadcasted_iota(jnp.int32, (NP, N_SC), 0)
    m1 = ((rank_row == r_col.astype(f32)) & (r_col < PRE)).astype(f32)
    r_row = jax.lax.broadcasted_iota(jnp.int32, (N_SC, NP), 1)
    m1t = ((rank_col == r_row.astype(f32)) & (r_row < PRE)).astype(f32)

    sel = jnp.dot(m1, d8_ref[...], preferred_element_type=f32)
    selt = jnp.dot(d8t_ref[...], m1t, preferred_element_type=f32)

    x1c, y1c, x2c, y2c = _decode_boxes(
        sel[:, 0:1], sel[:, 1:2], sel[:, 2:3], sel[:, 3:4],
        sel[:, 4:5], sel[:, 5:6], sel[:, 6:7], sel[:, 7:8])
    x1r, y1r, x2r, y2r = _decode_boxes(
        selt[0:1, :], selt[1:2, :], selt[2:3, :], selt[3:4, :],
        selt[4:5, :], selt[5:6, :], selt[6:7, :], selt[7:8, :])

    ltx = jnp.maximum(x1c, x1r)
    lty = jnp.maximum(y1c, y1r)
    rbx = jnp.minimum(x2c, x2r)
    rby = jnp.minimum(y2c, y2r)
    inter = jnp.maximum(rbx - ltx, 0.0) * jnp.maximum(rby - lty, 0.0)
    area_c = (x2c - x1c) * (y2c - y1c)
    area_r = (x2r - x1r) * (y2r - y1r)
    iou = inter / (area_c + area_r - inter + 1e-9)

    ii = jax.lax.broadcasted_iota(jnp.int32, (NP, NP), 0)
    jj = jax.lax.broadcasted_iota(jnp.int32, (NP, NP), 1)
    s_mat = ((iou > NMS_T) & (jj > ii) & (ii < PRE) & (jj < PRE)).astype(f32)

    valid = (jax.lax.broadcasted_iota(jnp.int32, (1, NP), 1) < PRE).astype(f32)

    def cond(carry):
        k, _, changed = carry
        return changed & (k < NP + 2)

    def body(carry):
        k, a, _ = carry
        supp = jnp.dot(a, s_mat, preferred_element_type=f32)
        a_new = jnp.where(supp > 0.0, 0.0, valid)
        changed = jnp.sum(jnp.abs(a_new - a)) > 0.0
        return k + 1, a_new, changed

    _, keep, _ = jax.lax.while_loop(
        cond, body, (jnp.int32(0), valid, jnp.bool_(True)))

    tri = (ii <= jj).astype(f32)
    cum = jnp.dot(keep, tri, preferred_element_type=f32)
    total = jnp.sum(keep)
    i_row = jax.lax.broadcasted_iota(jnp.int32, (1, NP), 1).astype(f32)
    rank2 = jnp.where(keep > 0.0, cum - 1.0, total + (i_row - cum))

    r2 = jax.lax.broadcasted_iota(jnp.int32, (128, NP), 0).astype(f32)
    p_sel = (rank2 == r2).astype(f32)
    boxes4 = jnp.concatenate([x1c, y1c, x2c, y2c], axis=1)
    o_ref[...] = jnp.dot(p_sel, boxes4, preferred_element_type=f32)


def kernel(x, w1, b1, w2, b2, w3, b3, wh, bh, wcls, bcls, wbox, bbox):
    xi = x.reshape(3, H_IMG, W_IMG)
    p1 = _im2col(xi, 2, ((0, 1), (0, 1)))
    y1 = _matmul(p1, w1.reshape(64, 27).T, b1, True)
    f1 = y1.reshape(128, 128, 64).transpose(2, 0, 1)
    p2 = _im2col(f1, 2, ((0, 1), (0, 1)))
    y2 = _matmul(p2, w2.reshape(128, 576).T, b2, True)
    f2 = y2.reshape(64, 64, 128).transpose(2, 0, 1)
    p3 = _im2col(f2, 2, ((0, 1), (0, 1)))
    y3 = _matmul(p3, w3.reshape(256, 1152).T, b3, True)
    f3 = y3.reshape(32, 32, 256).transpose(2, 0, 1)
    p4 = _im2col(f3, 1, ((1, 1), (1, 1)))
    y4 = _matmul(p4, wh.reshape(256, 2304).T, bh, True)

    wcb = jnp.concatenate([wcls.reshape(A, 256), wbox.reshape(A * 4, 256)],
                          axis=0).T
    bcb = jnp.concatenate([bcls, bbox], axis=0)
    y5 = _matmul(y4, wcb, bcb, False)

    scores = y5[:, :A].reshape(-1)
    d = y5[:, A:].reshape(HF * WF * A, 4)
    anchors = jnp.asarray(_make_anchors_np())
    data8 = jnp.concatenate([anchors, d], axis=1)

    out = pl.pallas_call(
        _proposal_body,
        out_shape=jax.ShapeDtypeStruct((128, 4), jnp.float32),
    )(scores.reshape(1, N_SC), scores.reshape(N_SC, 1), data8, data8.T)
    return out[:POST]


# speedup vs baseline: 10.3638x; 10.3638x over previous
import functools
import math

import jax
import jax.numpy as jnp
import numpy as np
from jax.experimental import pallas as pl

H_IMG = 256
W_IMG = 256
HF = 32
WF = 32
STRIDE = 8
A = 3
PRE = 1000
POST = 100
NMS_T = 0.7
N_SC = HF * WF * A
NP = 1024
DWH_CLIP = math.log(1000.0 / 16)


def _make_anchors_np():
    ratios = np.array([0.5, 1.0, 2.0])
    hr = np.sqrt(ratios)
    wr = 1.0 / hr
    size = float(min(HF, WF))
    ws = wr * size
    hs = hr * size
    base = np.stack([-ws / 2, -hs / 2, ws / 2, hs / 2], axis=1)
    sy, sx = np.meshgrid(np.arange(HF) * STRIDE, np.arange(WF) * STRIDE,
                         indexing='ij')
    shifts = np.stack([sx.ravel(), sy.ravel(), sx.ravel(), sy.ravel()], axis=1)
    anchors = (shifts[:, None, :] + base[None, :, :]).reshape(-1, 4)
    return anchors.astype(np.float32)


def _mm_body(a_ref, b_ref, bias_ref, o_ref, *, relu):
    acc = jnp.dot(a_ref[...], b_ref[...], preferred_element_type=jnp.float32)
    acc = acc + bias_ref[...]
    if relu:
        acc = jnp.maximum(acc, 0.0)
    o_ref[...] = acc


def _matmul(a, b, bias, relu, block_m=None):
    m, k = a.shape
    n = b.shape[1]
    if block_m is None or block_m >= m:
        return pl.pallas_call(
            functools.partial(_mm_body, relu=relu),
            out_shape=jax.ShapeDtypeStruct((m, n), jnp.float32),
        )(a, b, bias.reshape(1, n))
    return pl.pallas_call(
        functools.partial(_mm_body, relu=relu),
        grid=(m // block_m,),
        in_specs=[
            pl.BlockSpec((block_m, k), lambda i: (i, 0)),
            pl.BlockSpec((k, n), lambda i: (0, 0)),
            pl.BlockSpec((1, n), lambda i: (0, 0)),
        ],
        out_specs=pl.BlockSpec((block_m, n), lambda i: (i, 0)),
        out_shape=jax.ShapeDtypeStruct((m, n), jnp.float32),
    )(a, b, bias.reshape(1, n))


def _wt(w):
    return w.transpose(2, 3, 1, 0).reshape(-1, w.shape[0])


def _im2col(x, stride, pad):
    c, h, w = x.shape
    xp = jnp.pad(x, ((0, 0), pad[0], pad[1]))
    oh = (h + pad[0][0] + pad[0][1] - 3) // stride + 1
    ow = (w + pad[1][0] + pad[1][1] - 3) // stride + 1
    cols = []
    for ky in range(3):
        for kx in range(3):
            sl = jax.lax.slice(
                xp, (0, ky, kx),
                (c, ky + (oh - 1) * stride + 1, kx + (ow - 1) * stride + 1),
                (1, stride, stride))
            cols.append(sl)
    p = jnp.stack(cols, axis=0)
    p = p.transpose(2, 3, 0, 1)
    return p.reshape(oh * ow, 9 * c)


def _decode_boxes(ax1, ay1, ax2, ay2, dx, dy, dw, dh):
    aw = ax2 - ax1
    ah = ay2 - ay1
    acx = ax1 + 0.5 * aw
    acy = ay1 + 0.5 * ah
    dw = jnp.minimum(dw, DWH_CLIP)
    dh = jnp.minimum(dh, DWH_CLIP)
    pcx = dx * aw + acx
    pcy = dy * ah + acy
    pw = jnp.exp(dw) * aw
    ph = jnp.exp(dh) * ah
    x1 = jnp.clip(pcx - 0.5 * pw, 0.0, float(W_IMG))
    y1 = jnp.clip(pcy - 0.5 * ph, 0.0, float(H_IMG))
    x2 = jnp.clip(pcx + 0.5 * pw, 0.0, float(W_IMG))
    y2 = jnp.clip(pcy + 0.5 * ph, 0.0, float(H_IMG))
    return x1, y1, x2, y2


def _proposal_body(s_row_ref, s_col_ref, d8_ref, d8t_ref, o_ref):
    s_row = s_row_ref[...]
    s_col = s_col_ref[...]
    f32 = jnp.float32

    CH = 512
    rank_row = jnp.zeros((1, N_SC), f32)
    rank_col = jnp.zeros((N_SC, 1), f32)
    for c in range(N_SC // CH):
        sk_col = jax.lax.slice(s_col, (c * CH, 0), ((c + 1) * CH, 1))
        kk = jax.lax.broadcasted_iota(jnp.int32, (CH, N_SC), 0) + c * CH
        jj = jax.lax.broadcasted_iota(jnp.int32, (CH, N_SC), 1)
        beat = (sk_col > s_row) | ((sk_col == s_row) & (kk < jj))
        rank_row = rank_row + jnp.sum(beat.astype(f32), axis=0, keepdims=True)

        sk_row = jax.lax.slice(s_row, (0, c * CH), (1, (c + 1) * CH))
        kk2 = jax.lax.broadcasted_iota(jnp.int32, (N_SC, CH), 1) + c * CH
        ii2 = jax.lax.broadcasted_iota(jnp.int32, (N_SC, CH), 0)
        beat2 = (sk_row > s_col) | ((sk_row == s_col) & (kk2 < ii2))
        rank_col = rank_col + jnp.sum(beat2.astype(f32), axis=1, keepdims=True)

    r_col = jax.lax.broadcasted_iota(jnp.int32, (NP, N_SC), 0)
    m1 = ((rank_row == r_col.astype(f32)) & (r_col < PRE)).astype(f32)
    r_row = jax.lax.broadcasted_iota(jnp.int32, (N_SC, NP), 1)
    m1t = ((rank_col == r_row.astype(f32)) & (r_row < PRE)).astype(f32)

    sel = jnp.dot(m1, d8_ref[...], preferred_element_type=f32, precision=jax.lax.Precision.HIGHEST)
    selt = jnp.dot(d8t_ref[...], m1t, preferred_element_type=f32, precision=jax.lax.Precision.HIGHEST)

    x1c, y1c, x2c, y2c = _decode_boxes(
        sel[:, 0:1], sel[:, 1:2], sel[:, 2:3], sel[:, 3:4],
        sel[:, 4:5], sel[:, 5:6], sel[:, 6:7], sel[:, 7:8])
    x1r, y1r, x2r, y2r = _decode_boxes(
        selt[0:1, :], selt[1:2, :], selt[2:3, :], selt[3:4, :],
        selt[4:5, :], selt[5:6, :], selt[6:7, :], selt[7:8, :])

    ltx = jnp.maximum(x1c, x1r)
    lty = jnp.maximum(y1c, y1r)
    rbx = jnp.minimum(x2c, x2r)
    rby = jnp.minimum(y2c, y2r)
    inter = jnp.maximum(rbx - ltx, 0.0) * jnp.maximum(rby - lty, 0.0)
    area_c = (x2c - x1c) * (y2c - y1c)
    area_r = (x2r - x1r) * (y2r - y1r)
    iou = inter / (area_c + area_r - inter + 1e-9)

    ii = jax.lax.broadcasted_iota(jnp.int32, (NP, NP), 0)
    jj = jax.lax.broadcasted_iota(jnp.int32, (NP, NP), 1)
    s_mat = ((iou > NMS_T) & (jj > ii) & (ii < PRE) & (jj < PRE)).astype(f32)

    valid = (jax.lax.broadcasted_iota(jnp.int32, (1, NP), 1) < PRE).astype(f32)

    def cond(carry):
        k, _, changed = carry
        return changed & (k < NP + 2)

    def body(carry):
        k, a, _ = carry
        supp = jnp.dot(a, s_mat, preferred_element_type=f32, precision=jax.lax.Precision.HIGHEST)
        a_new = jnp.where(supp > 0.0, 0.0, valid)
        changed = jnp.sum(jnp.abs(a_new - a)) > 0.0
        return k + 1, a_new, changed

    _, keep, _ = jax.lax.while_loop(
        cond, body, (jnp.int32(0), valid, jnp.bool_(True)))

    tri = (ii <= jj).astype(f32)
    cum = jnp.dot(keep, tri, preferred_element_type=f32, precision=jax.lax.Precision.HIGHEST)
    total = jnp.sum(keep)
    i_row = jax.lax.broadcasted_iota(jnp.int32, (1, NP), 1).astype(f32)
    rank2 = jnp.where(keep > 0.0, cum - 1.0, total + (i_row - cum))

    r2 = jax.lax.broadcasted_iota(jnp.int32, (128, NP), 0).astype(f32)
    p_sel = (rank2 == r2).astype(f32)
    boxes4 = jnp.concatenate([x1c, y1c, x2c, y2c], axis=1)
    o_ref[...] = jnp.dot(p_sel, boxes4, preferred_element_type=f32, precision=jax.lax.Precision.HIGHEST)


def kernel(x, w1, b1, w2, b2, w3, b3, wh, bh, wcls, bcls, wbox, bbox):
    xi = x.reshape(3, H_IMG, W_IMG)
    p1 = _im2col(xi, 2, ((0, 1), (0, 1)))
    y1 = _matmul(p1, _wt(w1), b1, True, block_m=2048)
    f1 = y1.reshape(128, 128, 64).transpose(2, 0, 1)
    p2 = _im2col(f1, 2, ((0, 1), (0, 1)))
    y2 = _matmul(p2, _wt(w2), b2, True, block_m=1024)
    f2 = y2.reshape(64, 64, 128).transpose(2, 0, 1)
    p3 = _im2col(f2, 2, ((0, 1), (0, 1)))
    y3 = _matmul(p3, _wt(w3), b3, True, block_m=512)
    f3 = y3.reshape(32, 32, 256).transpose(2, 0, 1)
    p4 = _im2col(f3, 1, ((1, 1), (1, 1)))
    y4 = _matmul(p4, _wt(wh), bh, True, block_m=256)

    wcb = jnp.concatenate([wcls.reshape(A, 256), wbox.reshape(A * 4, 256)],
                          axis=0).T
    bcb = jnp.concatenate([bcls, bbox], axis=0)
    y5 = _matmul(y4, wcb, bcb, False)

    scores = y5[:, :A].reshape(-1)
    d = y5[:, A:].reshape(HF * WF * A, 4)
    anchors = jnp.asarray(_make_anchors_np())
    data8 = jnp.concatenate([anchors, d], axis=1)

    out = pl.pallas_call(
        _proposal_body,
        out_shape=jax.ShapeDtypeStruct((128, 4), jnp.float32),
    )(scores.reshape(1, N_SC), scores.reshape(N_SC, 1), data8, data8.T)
    return out[:POST]


# speedup vs baseline: 10.4223x; 1.0056x over previous
import functools
import math

import jax
import jax.numpy as jnp
import numpy as np
from jax.experimental import pallas as pl

H_IMG = 256
W_IMG = 256
HF = 32
WF = 32
STRIDE = 8
A = 3
PRE = 1000
POST = 100
NMS_T = 0.7
N_SC = HF * WF * A
NP = 1024
DWH_CLIP = math.log(1000.0 / 16)


def _make_anchors_np():
    ratios = np.array([0.5, 1.0, 2.0])
    hr = np.sqrt(ratios)
    wr = 1.0 / hr
    size = float(min(HF, WF))
    ws = wr * size
    hs = hr * size
    base = np.stack([-ws / 2, -hs / 2, ws / 2, hs / 2], axis=1)
    sy, sx = np.meshgrid(np.arange(HF) * STRIDE, np.arange(WF) * STRIDE,
                         indexing='ij')
    shifts = np.stack([sx.ravel(), sy.ravel(), sx.ravel(), sy.ravel()], axis=1)
    anchors = (shifts[:, None, :] + base[None, :, :]).reshape(-1, 4)
    return anchors.astype(np.float32)


def _mm_body(a_ref, b_ref, bias_ref, o_ref, *, relu):
    acc = jnp.dot(a_ref[...], b_ref[...], preferred_element_type=jnp.float32)
    acc = acc + bias_ref[...]
    if relu:
        acc = jnp.maximum(acc, 0.0)
    o_ref[...] = acc


def _matmul(a, b, bias, relu, block_m=None):
    m, k = a.shape
    n = b.shape[1]
    if block_m is None or block_m >= m:
        return pl.pallas_call(
            functools.partial(_mm_body, relu=relu),
            out_shape=jax.ShapeDtypeStruct((m, n), jnp.float32),
        )(a, b, bias.reshape(1, n))
    return pl.pallas_call(
        functools.partial(_mm_body, relu=relu),
        grid=(m // block_m,),
        in_specs=[
            pl.BlockSpec((block_m, k), lambda i: (i, 0)),
            pl.BlockSpec((k, n), lambda i: (0, 0)),
            pl.BlockSpec((1, n), lambda i: (0, 0)),
        ],
        out_specs=pl.BlockSpec((block_m, n), lambda i: (i, 0)),
        out_shape=jax.ShapeDtypeStruct((m, n), jnp.float32),
    )(a, b, bias.reshape(1, n))


def _wt(w):
    return w.transpose(2, 3, 1, 0).reshape(-1, w.shape[0])


def _im2col(x, stride, pad):
    c, h, w = x.shape
    xp = jnp.pad(x, ((0, 0), pad[0], pad[1]))
    oh = (h + pad[0][0] + pad[0][1] - 3) // stride + 1
    ow = (w + pad[1][0] + pad[1][1] - 3) // stride + 1
    cols = []
    for ky in range(3):
        for kx in range(3):
            sl = jax.lax.slice(
                xp, (0, ky, kx),
                (c, ky + (oh - 1) * stride + 1, kx + (ow - 1) * stride + 1),
                (1, stride, stride))
            cols.append(sl)
    p = jnp.stack(cols, axis=0)
    p = p.transpose(2, 3, 0, 1)
    return p.reshape(oh * ow, 9 * c)


def _decode_boxes(ax1, ay1, ax2, ay2, dx, dy, dw, dh):
    aw = ax2 - ax1
    ah = ay2 - ay1
    acx = ax1 + 0.5 * aw
    acy = ay1 + 0.5 * ah
    dw = jnp.minimum(dw, DWH_CLIP)
    dh = jnp.minimum(dh, DWH_CLIP)
    pcx = dx * aw + acx
    pcy = dy * ah + acy
    pw = jnp.exp(dw) * aw
    ph = jnp.exp(dh) * ah
    x1 = jnp.clip(pcx - 0.5 * pw, 0.0, float(W_IMG))
    y1 = jnp.clip(pcy - 0.5 * ph, 0.0, float(H_IMG))
    x2 = jnp.clip(pcx + 0.5 * pw, 0.0, float(W_IMG))
    y2 = jnp.clip(pcy + 0.5 * ph, 0.0, float(H_IMG))
    return x1, y1, x2, y2


def _proposal_body(s_row_ref, s_col_ref, d8_ref, d8t_ref, o_ref):
    s_row = s_row_ref[...]
    s_col = s_col_ref[...]
    f32 = jnp.float32

    CH = 512
    rank_row = jnp.zeros((1, N_SC), f32)
    rank_col = jnp.zeros((N_SC, 1), f32)
    for c in range(N_SC // CH):
        sk_col = jax.lax.slice(s_col, (c * CH, 0), ((c + 1) * CH, 1))
        kk = jax.lax.broadcasted_iota(jnp.int32, (CH, N_SC), 0) + c * CH
        jj = jax.lax.broadcasted_iota(jnp.int32, (CH, N_SC), 1)
        beat = (sk_col > s_row) | ((sk_col == s_row) & (kk < jj))
        rank_row = rank_row + jnp.sum(beat.astype(f32), axis=0, keepdims=True)

        sk_row = jax.lax.slice(s_row, (0, c * CH), (1, (c + 1) * CH))
        kk2 = jax.lax.broadcasted_iota(jnp.int32, (N_SC, CH), 1) + c * CH
        ii2 = jax.lax.broadcasted_iota(jnp.int32, (N_SC, CH), 0)
        beat2 = (sk_row > s_col) | ((sk_row == s_col) & (kk2 < ii2))
        rank_col = rank_col + jnp.sum(beat2.astype(f32), axis=1, keepdims=True)

    r_col = jax.lax.broadcasted_iota(jnp.int32, (NP, N_SC), 0)
    m1 = ((rank_row == r_col.astype(f32)) & (r_col < PRE)).astype(f32)
    r_row = jax.lax.broadcasted_iota(jnp.int32, (N_SC, NP), 1)
    m1t = ((rank_col == r_row.astype(f32)) & (r_row < PRE)).astype(f32)

    sel = jnp.dot(m1, d8_ref[...], preferred_element_type=f32, precision=jax.lax.Precision.HIGHEST)
    selt = jnp.dot(d8t_ref[...], m1t, preferred_element_type=f32, precision=jax.lax.Precision.HIGHEST)

    x1c, y1c, x2c, y2c = _decode_boxes(
        sel[:, 0:1], sel[:, 1:2], sel[:, 2:3], sel[:, 3:4],
        sel[:, 4:5], sel[:, 5:6], sel[:, 6:7], sel[:, 7:8])
    x1r, y1r, x2r, y2r = _decode_boxes(
        selt[0:1, :], selt[1:2, :], selt[2:3, :], selt[3:4, :],
        selt[4:5, :], selt[5:6, :], selt[6:7, :], selt[7:8, :])

    ltx = jnp.maximum(x1c, x1r)
    lty = jnp.maximum(y1c, y1r)
    rbx = jnp.minimum(x2c, x2r)
    rby = jnp.minimum(y2c, y2r)
    inter = jnp.maximum(rbx - ltx, 0.0) * jnp.maximum(rby - lty, 0.0)
    area_c = (x2c - x1c) * (y2c - y1c)
    area_r = (x2r - x1r) * (y2r - y1r)
    iou = inter / (area_c + area_r - inter + 1e-9)

    ii = jax.lax.broadcasted_iota(jnp.int32, (NP, NP), 0)
    jj = jax.lax.broadcasted_iota(jnp.int32, (NP, NP), 1)
    s_mat = ((iou > NMS_T) & (jj > ii) & (ii < PRE) & (jj < PRE)).astype(f32)

    valid = (jax.lax.broadcasted_iota(jnp.int32, (1, NP), 1) < PRE).astype(f32)

    def cond(carry):
        k, _, changed = carry
        return changed & (k < NP + 8)

    def body(carry):
        k, a, _ = carry
        for _ in range(3):
            supp = jnp.dot(a, s_mat, preferred_element_type=f32)
            a = jnp.where(supp > 0.0, 0.0, valid)
        supp = jnp.dot(a, s_mat, preferred_element_type=f32)
        a_new = jnp.where(supp > 0.0, 0.0, valid)
        changed = jnp.sum(jnp.abs(a_new - a)) > 0.0
        return k + 4, a_new, changed

    _, keep, _ = jax.lax.while_loop(
        cond, body, (jnp.int32(0), valid, jnp.bool_(True)))

    tri = (ii <= jj).astype(f32)
    cum = jnp.dot(keep, tri, preferred_element_type=f32)
    total = jnp.sum(keep)
    i_row = jax.lax.broadcasted_iota(jnp.int32, (1, NP), 1).astype(f32)
    rank2 = jnp.where(keep > 0.0, cum - 1.0, total + (i_row - cum))

    r2 = jax.lax.broadcasted_iota(jnp.int32, (128, NP), 0).astype(f32)
    p_sel = (rank2 == r2).astype(f32)
    boxes4 = jnp.concatenate([x1c, y1c, x2c, y2c], axis=1)
    o_ref[...] = jnp.dot(p_sel, boxes4, preferred_element_type=f32, precision=jax.lax.Precision.HIGHEST)


def kernel(x, w1, b1, w2, b2, w3, b3, wh, bh, wcls, bcls, wbox, bbox):
    xi = x.reshape(3, H_IMG, W_IMG)
    p1 = _im2col(xi, 2, ((0, 1), (0, 1)))
    y1 = _matmul(p1, _wt(w1), b1, True, block_m=2048)
    f1 = y1.reshape(128, 128, 64).transpose(2, 0, 1)
    p2 = _im2col(f1, 2, ((0, 1), (0, 1)))
    y2 = _matmul(p2, _wt(w2), b2, True, block_m=1024)
    f2 = y2.reshape(64, 64, 128).transpose(2, 0, 1)
    p3 = _im2col(f2, 2, ((0, 1), (0, 1)))
    y3 = _matmul(p3, _wt(w3), b3, True, block_m=512)
    f3 = y3.reshape(32, 32, 256).transpose(2, 0, 1)
    p4 = _im2col(f3, 1, ((1, 1), (1, 1)))
    y4 = _matmul(p4, _wt(wh), bh, True, block_m=256)

    wcb = jnp.concatenate([wcls.reshape(A, 256), wbox.reshape(A * 4, 256)],
                          axis=0).T
    bcb = jnp.concatenate([bcls, bbox], axis=0)
    y5 = _matmul(y4, wcb, bcb, False)

    scores = y5[:, :A].reshape(-1)
    d = y5[:, A:].reshape(HF * WF * A, 4)
    anchors = jnp.asarray(_make_anchors_np())
    data8 = jnp.concatenate([anchors, d], axis=1)

    out = pl.pallas_call(
        _proposal_body,
        out_shape=jax.ShapeDtypeStruct((128, 4), jnp.float32),
    )(scores.reshape(1, N_SC), scores.reshape(N_SC, 1), data8, data8.T)
    return out[:POST]


# speedup vs baseline: 10.5154x; 1.0089x over previous
import functools
import math

import jax
import jax.numpy as jnp
import numpy as np
from jax.experimental import pallas as pl

H_IMG = 256
W_IMG = 256
HF = 32
WF = 32
STRIDE = 8
A = 3
PRE = 1000
POST = 100
NMS_T = 0.7
N_SC = HF * WF * A
NP = 1024
DWH_CLIP = math.log(1000.0 / 16)


def _make_anchors_np():
    ratios = np.array([0.5, 1.0, 2.0])
    hr = np.sqrt(ratios)
    wr = 1.0 / hr
    size = float(min(HF, WF))
    ws = wr * size
    hs = hr * size
    base = np.stack([-ws / 2, -hs / 2, ws / 2, hs / 2], axis=1)
    sy, sx = np.meshgrid(np.arange(HF) * STRIDE, np.arange(WF) * STRIDE,
                         indexing='ij')
    shifts = np.stack([sx.ravel(), sy.ravel(), sx.ravel(), sy.ravel()], axis=1)
    anchors = (shifts[:, None, :] + base[None, :, :]).reshape(-1, 4)
    return anchors.astype(np.float32)


def _mm_body(a_ref, b_ref, bias_ref, o_ref, *, relu):
    acc = jnp.dot(a_ref[...], b_ref[...], preferred_element_type=jnp.float32)
    acc = acc + bias_ref[...]
    if relu:
        acc = jnp.maximum(acc, 0.0)
    o_ref[...] = acc


def _matmul(a, b, bias, relu, block_m=None):
    m, k = a.shape
    n = b.shape[1]
    if block_m is None or block_m >= m:
        return pl.pallas_call(
            functools.partial(_mm_body, relu=relu),
            out_shape=jax.ShapeDtypeStruct((m, n), jnp.float32),
        )(a, b, bias.reshape(1, n))
    return pl.pallas_call(
        functools.partial(_mm_body, relu=relu),
        grid=(m // block_m,),
        in_specs=[
            pl.BlockSpec((block_m, k), lambda i: (i, 0)),
            pl.BlockSpec((k, n), lambda i: (0, 0)),
            pl.BlockSpec((1, n), lambda i: (0, 0)),
        ],
        out_specs=pl.BlockSpec((block_m, n), lambda i: (i, 0)),
        out_shape=jax.ShapeDtypeStruct((m, n), jnp.float32),
    )(a, b, bias.reshape(1, n))


def _wt(w):
    return w.transpose(2, 3, 1, 0).reshape(-1, w.shape[0])


def _im2col(y, h, w, stride, pad):
    c = y.shape[1]
    x = y.reshape(h, w, c)
    xp = jnp.pad(x, (pad[0], pad[1], (0, 0)))
    oh = (h + pad[0][0] + pad[0][1] - 3) // stride + 1
    ow = (w + pad[1][0] + pad[1][1] - 3) // stride + 1
    taps = []
    for ky in range(3):
        for kx in range(3):
            sl = jax.lax.slice(
                xp, (ky, kx, 0),
                (ky + (oh - 1) * stride + 1, kx + (ow - 1) * stride + 1, c),
                (stride, stride, 1))
            taps.append(sl[:, :, None, :])
    p = jnp.concatenate(taps, axis=2)
    return p.reshape(oh * ow, 9 * c)


def _decode_boxes(ax1, ay1, ax2, ay2, dx, dy, dw, dh):
    aw = ax2 - ax1
    ah = ay2 - ay1
    acx = ax1 + 0.5 * aw
    acy = ay1 + 0.5 * ah
    dw = jnp.minimum(dw, DWH_CLIP)
    dh = jnp.minimum(dh, DWH_CLIP)
    pcx = dx * aw + acx
    pcy = dy * ah + acy
    pw = jnp.exp(dw) * aw
    ph = jnp.exp(dh) * ah
    x1 = jnp.clip(pcx - 0.5 * pw, 0.0, float(W_IMG))
    y1 = jnp.clip(pcy - 0.5 * ph, 0.0, float(H_IMG))
    x2 = jnp.clip(pcx + 0.5 * pw, 0.0, float(W_IMG))
    y2 = jnp.clip(pcy + 0.5 * ph, 0.0, float(H_IMG))
    return x1, y1, x2, y2


def _proposal_body(s_row_ref, s_col_ref, d8_ref, d8t_ref, o_ref):
    s_row = s_row_ref[...]
    s_col = s_col_ref[...]
    f32 = jnp.float32

    CH = 512
    rank_row = jnp.zeros((1, N_SC), f32)
    rank_col = jnp.zeros((N_SC, 1), f32)
    for c in range(N_SC // CH):
        sk_col = jax.lax.slice(s_col, (c * CH, 0), ((c + 1) * CH, 1))
        kk = jax.lax.broadcasted_iota(jnp.int32, (CH, N_SC), 0) + c * CH
        jj = jax.lax.broadcasted_iota(jnp.int32, (CH, N_SC), 1)
        beat = (sk_col > s_row) | ((sk_col == s_row) & (kk < jj))
        rank_row = rank_row + jnp.sum(beat.astype(f32), axis=0, keepdims=True)

        sk_row = jax.lax.slice(s_row, (0, c * CH), (1, (c + 1) * CH))
        kk2 = jax.lax.broadcasted_iota(jnp.int32, (N_SC, CH), 1) + c * CH
        ii2 = jax.lax.broadcasted_iota(jnp.int32, (N_SC, CH), 0)
        beat2 = (sk_row > s_col) | ((sk_row == s_col) & (kk2 < ii2))
        rank_col = rank_col + jnp.sum(beat2.astype(f32), axis=1, keepdims=True)

    r_col = jax.lax.broadcasted_iota(jnp.int32, (NP, N_SC), 0)
    m1 = ((rank_row == r_col.astype(f32)) & (r_col < PRE)).astype(f32)
    r_row = jax.lax.broadcasted_iota(jnp.int32, (N_SC, NP), 1)
    m1t = ((rank_col == r_row.astype(f32)) & (r_row < PRE)).astype(f32)

    sel = jnp.dot(m1, d8_ref[...], preferred_element_type=f32, precision=jax.lax.Precision.HIGHEST)
    selt = jnp.dot(d8t_ref[...], m1t, preferred_element_type=f32, precision=jax.lax.Precision.HIGHEST)

    x1c, y1c, x2c, y2c = _decode_boxes(
        sel[:, 0:1], sel[:, 1:2], sel[:, 2:3], sel[:, 3:4],
        sel[:, 4:5], sel[:, 5:6], sel[:, 6:7], sel[:, 7:8])
    x1r, y1r, x2r, y2r = _decode_boxes(
        selt[0:1, :], selt[1:2, :], selt[2:3, :], selt[3:4, :],
        selt[4:5, :], selt[5:6, :], selt[6:7, :], selt[7:8, :])

    ltx = jnp.maximum(x1c, x1r)
    lty = jnp.maximum(y1c, y1r)
    rbx = jnp.minimum(x2c, x2r)
    rby = jnp.minimum(y2c, y2r)
    inter = jnp.maximum(rbx - ltx, 0.0) * jnp.maximum(rby - lty, 0.0)
    area_c = (x2c - x1c) * (y2c - y1c)
    area_r = (x2r - x1r) * (y2r - y1r)
    iou = inter / (area_c + area_r - inter + 1e-9)

    ii = jax.lax.broadcasted_iota(jnp.int32, (NP, NP), 0)
    jj = jax.lax.broadcasted_iota(jnp.int32, (NP, NP), 1)
    s_mat = ((iou > NMS_T) & (jj > ii) & (ii < PRE) & (jj < PRE)).astype(f32)

    valid = (jax.lax.broadcasted_iota(jnp.int32, (1, NP), 1) < PRE).astype(f32)

    def cond(carry):
        k, _, changed = carry
        return changed & (k < NP + 8)

    def body(carry):
        k, a, _ = carry
        for _ in range(3):
            supp = jnp.dot(a, s_mat, preferred_element_type=f32)
            a = jnp.where(supp > 0.0, 0.0, valid)
        supp = jnp.dot(a, s_mat, preferred_element_type=f32)
        a_new = jnp.where(supp > 0.0, 0.0, valid)
        changed = jnp.sum(jnp.abs(a_new - a)) > 0.0
        return k + 4, a_new, changed

    _, keep, _ = jax.lax.while_loop(
        cond, body, (jnp.int32(0), valid, jnp.bool_(True)))

    tri = (ii <= jj).astype(f32)
    cum = jnp.dot(keep, tri, preferred_element_type=f32)
    total = jnp.sum(keep)
    i_row = jax.lax.broadcasted_iota(jnp.int32, (1, NP), 1).astype(f32)
    rank2 = jnp.where(keep > 0.0, cum - 1.0, total + (i_row - cum))

    r2 = jax.lax.broadcasted_iota(jnp.int32, (128, NP), 0).astype(f32)
    p_sel = (rank2 == r2).astype(f32)
    boxes4 = jnp.concatenate([x1c, y1c, x2c, y2c], axis=1)
    o_ref[...] = jnp.dot(p_sel, boxes4, preferred_element_type=f32, precision=jax.lax.Precision.HIGHEST)


def kernel(x, w1, b1, w2, b2, w3, b3, wh, bh, wcls, bcls, wbox, bbox):
    xi = x.reshape(3, H_IMG, W_IMG).transpose(1, 2, 0).reshape(-1, 3)
    p1 = _im2col(xi, H_IMG, W_IMG, 2, ((0, 1), (0, 1)))
    y1 = _matmul(p1, _wt(w1), b1, True, block_m=2048)
    p2 = _im2col(y1, 128, 128, 2, ((0, 1), (0, 1)))
    y2 = _matmul(p2, _wt(w2), b2, True, block_m=1024)
    p3 = _im2col(y2, 64, 64, 2, ((0, 1), (0, 1)))
    y3 = _matmul(p3, _wt(w3), b3, True, block_m=512)
    p4 = _im2col(y3, 32, 32, 1, ((1, 1), (1, 1)))
    y4 = _matmul(p4, _wt(wh), bh, True, block_m=256)

    wcb = jnp.concatenate([wcls.reshape(A, 256), wbox.reshape(A * 4, 256)],
                          axis=0).T
    bcb = jnp.concatenate([bcls, bbox], axis=0)
    y5 = _matmul(y4, wcb, bcb, False)

    scores = y5[:, :A].reshape(-1)
    d = y5[:, A:].reshape(HF * WF * A, 4)
    anchors = jnp.asarray(_make_anchors_np())
    data8 = jnp.concatenate([anchors, d], axis=1)

    out = pl.pallas_call(
        _proposal_body,
        out_shape=jax.ShapeDtypeStruct((128, 4), jnp.float32),
    )(scores.reshape(1, N_SC), scores.reshape(N_SC, 1), data8, data8.T)
    return out[:POST]
